# baseline (speedup 1.0000x reference)
import jax
import jax.numpy as jnp
from jax import lax
from jax.experimental import pallas as pl
from jax.experimental.pallas import tpu as pltpu
from jax.experimental.pallas import tpu_sc as plsc

M_ITEM = 1000000
DIM = 16
N_GRAD = 16384
LR = 0.01

NW = 32
RPT = M_ITEM // NW
CH = 625
NCH = RPT // CH
NSLOT = 3
CAP = 3072
GB = 128
SCAN_IT = N_GRAD // 16


def _body(w_hbm, g_hbm, it_hbm, out_hbm,
          items_v, posm_v, idxm_v, updm_v, posc_v, buf,
          sem_in, sem_out, sem_g):
    wid = lax.axis_index("s") * 2 + lax.axis_index("c")
    base = wid * RPT

    def chunk_src(c):
        return w_hbm.at[pl.ds(base + c * CH, CH), :]

    def chunk_dst(c):
        return out_hbm.at[pl.ds(base + c * CH, CH), :]

    pltpu.async_copy(chunk_src(0), buf.at[0], sem_in)
    pltpu.async_copy(chunk_src(1), buf.at[1], sem_in)

    pltpu.sync_copy(it_hbm, items_v)

    def zero_body(i, _):
        posm_v[pl.ds(i * 16, 16)] = jnp.zeros((16,), jnp.int32)
        return 0
    lax.fori_loop(0, (CAP + GB) // 16, zero_body, 0)

    lanes = lax.iota(jnp.int32, 16)

    def scan_body(i, cnt):
        v = items_v[pl.ds(i * 16, 16)]
        msk = (v >= base) & (v < base + RPT)
        off = jnp.minimum(cnt, CAP)
        pf = plsc.cumsum(jnp.where(msk, jnp.int32(1), jnp.int32(0)))
        dst = off + pf - 1
        plsc.store_scatter(posm_v, [dst], lanes + i * 16, mask=msk)
        plsc.store_scatter(idxm_v, [dst], v - base, mask=msk)
        return cnt + pf[15]
    cnt = lax.fori_loop(0, SCAN_IT, scan_body, jnp.int32(0))
    cnt = jnp.minimum(cnt, CAP)

    nb = (cnt + (GB - 1)) // GB

    def gather_body(bi, _):
        pltpu.async_copy(
            g_hbm.at[posm_v.at[pl.ds(bi * GB, GB)]],
            updm_v.at[pl.ds(bi * GB, GB), :], sem_g).wait()
        return 0
    lax.fori_loop(0, nb, gather_body, 0)

    magic = jnp.int32(0x5F3759DF)

    def clip_body(j, _):
        row = updm_v[j, :]
        ssq = jnp.sum(row * row)
        x = jnp.maximum(lax.broadcast_in_dim(ssq, (16,), ()),
                        jnp.float32(1e-24))
        y = plsc.bitcast(magic - (plsc.bitcast(x, jnp.int32) >> 1),
                         jnp.float32)
        y = y * (1.5 - 0.5 * x * y * y)
        y = y * (1.5 - 0.5 * x * y * y)
        y = y * (1.5 - 0.5 * x * y * y)
        scale = jnp.minimum(jnp.float32(1.0), y) * jnp.float32(-LR)
        updm_v[j, :] = row * scale
        return 0
    lax.fori_loop(0, cnt, clip_body, 0)

    def chunk_body(c, _):
        b = lax.rem(c, NSLOT)
        pltpu.make_async_copy(chunk_src(c), buf.at[b], sem_in).wait()

        def sub_body(t, k):
            lv = idxm_v[pl.ds(t * 16, 16)]
            jv = lanes + t * 16
            msk = (lv >= c * CH) & (lv < (c + 1) * CH) & (jv < cnt)
            pf = plsc.cumsum(jnp.where(msk, jnp.int32(1), jnp.int32(0)))
            plsc.store_scatter(posc_v, [k + pf - 1], jv, mask=msk)
            return k + pf[15]
        nmatch = lax.fori_loop(0, (cnt + 15) // 16, sub_body, jnp.int32(0))

        def app_body(j, _):
            p = posc_v[pl.ds(j, 16)][0]
            li = idxm_v[pl.ds(p, 16)][0] - c * CH
            buf[b, li, :] = buf[b, li, :] + updm_v[p, :]
            return 0
        lax.fori_loop(0, nmatch, app_body, 0)

        pltpu.async_copy(buf.at[b], chunk_dst(c), sem_out)

        @pl.when(c >= 1)
        def _():
            bp = lax.rem(c - 1, NSLOT)
            pltpu.make_async_copy(buf.at[bp], chunk_dst(c - 1), sem_out).wait()

        @pl.when(c + 2 < NCH)
        def _():
            bn = lax.rem(c + 2, NSLOT)
            pltpu.async_copy(chunk_src(c + 2), buf.at[bn], sem_in)
        return 0
    lax.fori_loop(0, NCH, chunk_body, 0)

    pltpu.make_async_copy(buf.at[(NCH - 1) % NSLOT], chunk_dst(NCH - 1),
                          sem_out).wait()


def kernel(items_emb_weight, items_emb_grad, items):
    run = pl.kernel(
        _body,
        out_type=jax.ShapeDtypeStruct((M_ITEM, DIM), jnp.float32),
        mesh=plsc.VectorSubcoreMesh(core_axis_name="c", subcore_axis_name="s"),
        compiler_params=pltpu.CompilerParams(use_tc_tiling_on_sc=False,
                                             needs_layout_passes=False),
        scratch_types=[
            pltpu.VMEM((N_GRAD,), jnp.int32),
            pltpu.VMEM((CAP + GB,), jnp.int32),
            pltpu.VMEM((CAP + 16,), jnp.int32),
            pltpu.VMEM((CAP, DIM), jnp.float32),
            pltpu.VMEM((CAP + 16,), jnp.int32),
            pltpu.VMEM((NSLOT, CH, DIM), jnp.float32),
            pltpu.SemaphoreType.DMA,
            pltpu.SemaphoreType.DMA,
            pltpu.SemaphoreType.DMA,
        ],
    )
    return run(items_emb_weight, items_emb_grad,
               items.astype(jnp.int32))


# speedup vs baseline: 4.0220x; 4.0220x over previous
import jax
import jax.numpy as jnp
from jax import lax
from jax.experimental import pallas as pl
from jax.experimental.pallas import tpu as pltpu
from jax.experimental.pallas import tpu_sc as plsc

M_ITEM = 1000000
DIM = 16
N_GRAD = 16384
LR = 0.01

NW = 32
UT = 128
U_PER_TILE = 244
ICH = 512
NMAIN = U_PER_TILE * UT // ICH
I_MAIN = NMAIN * ICH
TAIL_START = 999936
NSLOT = 3
CAP = 3072
GB = 128
SCAN_IT = N_GRAD // 16


def _body(w_hbm, g_hbm, it_hbm, out_hbm,
          items_v, posm_v, idxm_v, updm_v, posc_v, gidx_v, gbuf_v, buf,
          xbuf, xbuf2, sem_in, sem_out, sem_g):
    wid = lax.axis_index("s") * 2 + lax.axis_index("c")
    ibase = (wid * U_PER_TILE + jnp.minimum(wid, 4)) * UT
    n_main = I_MAIN + jnp.where(wid < 4, UT, 0)

    def chunk_src(c):
        return w_hbm.at[:, pl.ds(ibase + c * ICH, ICH)]

    def chunk_dst(c):
        return out_hbm.at[:, pl.ds(ibase + c * ICH, ICH)]

    pltpu.async_copy(chunk_src(0), buf.at[0], sem_in)
    pltpu.async_copy(chunk_src(1), buf.at[1], sem_in)

    pltpu.sync_copy(it_hbm, items_v)

    def zero_body(i, _):
        posm_v[pl.ds(i * 16, 16)] = jnp.zeros((16,), jnp.int32)
        return 0
    lax.fori_loop(0, (CAP + 16) // 16, zero_body, 0)

    lanes = lax.iota(jnp.int32, 16)

    def scan_body(i, cnt):
        v = items_v[pl.ds(i * 16, 16)]
        in_main = (v >= ibase) & (v < ibase + n_main)
        in_tail = (v >= TAIL_START) & (wid == 4)
        msk = in_main | in_tail
        lv = jnp.where(in_tail, I_MAIN + (v - TAIL_START), v - ibase)
        off = jnp.minimum(cnt, CAP)
        pf = plsc.cumsum(jnp.where(msk, jnp.int32(1), jnp.int32(0)))
        dst = off + pf - 1
        plsc.store_scatter(posm_v, [dst], lanes + i * 16, mask=msk)
        plsc.store_scatter(idxm_v, [dst], lv, mask=msk)
        return cnt + pf[15]
    cnt = lax.fori_loop(0, SCAN_IT, scan_body, jnp.int32(0))
    cnt = jnp.minimum(cnt, CAP)

    nb = (cnt + (GB - 1)) // GB

    def gather_body(bi, _):
        def stage(k, _):
            pv = posm_v[pl.ds(bi * GB + k * 16, 16)]
            gidx_v[pl.ds(k * 16, 16)] = pv >> 3
            return 0
        lax.fori_loop(0, GB // 16, stage, 0)
        pltpu.async_copy(g_hbm.at[gidx_v], gbuf_v, sem_g).wait()

        def compact(r, _):
            p = posm_v[pl.ds(bi * GB + r, 16)][0]
            sub = (p & 7) * DIM
            updm_v[pl.ds((bi * GB + r) * DIM, 16)] = gbuf_v[r, pl.ds(sub, 16)]
            return 0
        lax.fori_loop(0, GB, compact, 0)
        return 0
    lax.fori_loop(0, nb, gather_body, 0)

    magic = jnp.int32(0x5F3759DF)

    def clip_body(j, _):
        row = updm_v[pl.ds(j * DIM, 16)]
        ssq = jnp.sum(row * row)
        x = jnp.maximum(lax.broadcast_in_dim(ssq, (16,), ()),
                        jnp.float32(1e-24))
        y = plsc.bitcast(magic - (plsc.bitcast(x, jnp.int32) >> 1),
                         jnp.float32)
        y = y * (1.5 - 0.5 * x * y * y)
        y = y * (1.5 - 0.5 * x * y * y)
        y = y * (1.5 - 0.5 * x * y * y)
        scale = jnp.minimum(jnp.float32(1.0), y) * jnp.float32(-LR)
        updm_v[pl.ds(j * DIM, 16)] = row * scale
        return 0
    lax.fori_loop(0, cnt, clip_body, 0)

    nscan16 = (cnt + 15) // 16

    def build_sublist(lo, hi):
        def sub_body(t, k):
            lv = idxm_v[pl.ds(t * 16, 16)]
            jv = lanes + t * 16
            msk = (lv >= lo) & (lv < hi) & (jv < cnt)
            pf = plsc.cumsum(jnp.where(msk, jnp.int32(1), jnp.int32(0)))
            plsc.store_scatter(posc_v, [k + pf - 1], jv, mask=msk)
            return k + pf[15]
        return lax.fori_loop(0, nscan16, sub_body, jnp.int32(0))

    def apply_sublist(nmatch, lo, dst_ref):
        def app_body(j, _):
            p = posc_v[pl.ds(j, 16)][0]
            il = idxm_v[pl.ds(p, 16)][0] - lo
            ilv = lax.broadcast_in_dim(il, (16,), ())
            plsc.addupdate_scatter(dst_ref, [lanes, ilv],
                                   updm_v[pl.ds(p * DIM, 16)])
            return 0
        lax.fori_loop(0, nmatch, app_body, 0)

    def chunk_body(c, _):
        b = lax.rem(c, NSLOT)
        pltpu.make_async_copy(chunk_src(c), buf.at[b], sem_in).wait()
        nmatch = build_sublist(c * ICH, (c + 1) * ICH)
        apply_sublist(nmatch, c * ICH, buf.at[b])
        pltpu.async_copy(buf.at[b], chunk_dst(c), sem_out)

        @pl.when(c >= 1)
        def _():
            bp = lax.rem(c - 1, NSLOT)
            pltpu.make_async_copy(buf.at[bp], chunk_dst(c - 1), sem_out).wait()

        @pl.when(c + 2 < NMAIN)
        def _():
            bn = lax.rem(c + 2, NSLOT)
            pltpu.async_copy(chunk_src(c + 2), buf.at[bn], sem_in)
        return 0
    lax.fori_loop(0, NMAIN, chunk_body, 0)

    pltpu.make_async_copy(buf.at[(NMAIN - 1) % NSLOT], chunk_dst(NMAIN - 1),
                          sem_out).wait()

    @pl.when(wid < 4)
    def _():
        xs = pl.ds(ibase + I_MAIN, UT)
        pltpu.sync_copy(w_hbm.at[:, xs], xbuf)
        nmatch = build_sublist(I_MAIN, I_MAIN + UT)
        apply_sublist(nmatch, I_MAIN, xbuf)
        pltpu.sync_copy(xbuf, out_hbm.at[:, xs])

    @pl.when(wid == 4)
    def _():
        xs = pl.ds(TAIL_START, 64)
        pltpu.sync_copy(w_hbm.at[:, xs], xbuf2)
        nmatch = build_sublist(I_MAIN, I_MAIN + 64)
        apply_sublist(nmatch, I_MAIN, xbuf2)
        pltpu.sync_copy(xbuf2, out_hbm.at[:, xs])


def kernel(items_emb_weight, items_emb_grad, items):
    run = pl.kernel(
        _body,
        out_type=jax.ShapeDtypeStruct((DIM, M_ITEM), jnp.float32),
        mesh=plsc.VectorSubcoreMesh(core_axis_name="c", subcore_axis_name="s"),
        compiler_params=pltpu.CompilerParams(use_tc_tiling_on_sc=True,
                                             needs_layout_passes=False),
        scratch_types=[
            pltpu.VMEM((N_GRAD,), jnp.int32),
            pltpu.VMEM((CAP + 16,), jnp.int32),
            pltpu.VMEM((CAP + 16,), jnp.int32),
            pltpu.VMEM((CAP * DIM,), jnp.float32),
            pltpu.VMEM((CAP + 16,), jnp.int32),
            pltpu.VMEM((GB,), jnp.int32),
            pltpu.VMEM((GB, 128), jnp.float32),
            pltpu.VMEM((NSLOT, DIM, ICH), jnp.float32),
            pltpu.VMEM((DIM, UT), jnp.float32),
            pltpu.VMEM((DIM, 64), jnp.float32),
            pltpu.SemaphoreType.DMA,
            pltpu.SemaphoreType.DMA,
            pltpu.SemaphoreType.DMA,
        ],
    )
    out = run(items_emb_weight.T,
              items_emb_grad.reshape(N_GRAD // 8, 8 * DIM),
              items.astype(jnp.int32))
    return out.T


# speedup vs baseline: 4.0425x; 1.0051x over previous
import jax
import jax.numpy as jnp
from jax import lax
from jax.experimental import pallas as pl
from jax.experimental.pallas import tpu as pltpu
from jax.experimental.pallas import tpu_sc as plsc

M_ITEM = 1000000
DIM = 16
N_GRAD = 16384
LR = 0.01

NW = 32
UT = 128
U_PER_TILE = 244
ICH = 512
NMAIN = U_PER_TILE * UT // ICH
I_MAIN = NMAIN * ICH
TAIL_START = 999936
NSLOT = 3
CAP = 3072
GB = 128
SCAN_IT = N_GRAD // 16


def _body(w_hbm, g_hbm, it_hbm, out_hbm,
          items_v, posm_v, idxm_v, updm_v, posc_v, gidx_v, gbuf_v, buf,
          xbuf, xbuf2, sem_in, sem_out, sem_g):
    wid = lax.axis_index("s") * 2 + lax.axis_index("c")
    ibase = (wid * U_PER_TILE + jnp.minimum(wid, 4)) * UT
    n_main = I_MAIN + jnp.where(wid < 4, UT, 0)

    def chunk_src(c):
        return w_hbm.at[:, pl.ds(ibase + c * ICH, ICH)]

    def chunk_dst(c):
        return out_hbm.at[:, pl.ds(ibase + c * ICH, ICH)]

    pltpu.async_copy(chunk_src(0), buf.at[0], sem_in.at[0])
    pltpu.async_copy(chunk_src(1), buf.at[1], sem_in.at[1])

    pltpu.sync_copy(it_hbm, items_v)

    def zero_body(i, _):
        gidx_v[pl.ds(i * 16, 16)] = jnp.zeros((16,), jnp.int32)
        return 0
    lax.fori_loop(0, (CAP + GB) // 16, zero_body, 0)

    lanes = lax.iota(jnp.int32, 16)

    def scan_body(i, cnt):
        v = items_v[pl.ds(i * 16, 16)]
        in_main = (v >= ibase) & (v < ibase + n_main)
        in_tail = (v >= TAIL_START) & (wid == 4)
        msk = in_main | in_tail
        lv = jnp.where(in_tail, I_MAIN + (v - TAIL_START), v - ibase)
        off = jnp.minimum(cnt, CAP)
        pf = plsc.cumsum(jnp.where(msk, jnp.int32(1), jnp.int32(0)))
        dst = off + pf - 1
        pos = lanes + i * 16
        plsc.store_scatter(posm_v, [dst], pos, mask=msk)
        plsc.store_scatter(idxm_v, [dst], lv, mask=msk)
        plsc.store_scatter(gidx_v, [dst], pos >> 3, mask=msk)
        return cnt + pf[15]
    cnt = lax.fori_loop(0, SCAN_IT, scan_body, jnp.int32(0))
    cnt = jnp.minimum(cnt, CAP)

    nb = (cnt + (GB - 1)) // GB

    def gather_body(bi, _):
        pltpu.async_copy(g_hbm.at[gidx_v.at[pl.ds(bi * GB, GB)]], gbuf_v,
                         sem_g).wait()

        def compact(r, _):
            p = posm_v[pl.ds(bi * GB + r, 16)][0]
            sub = (p & 7) * DIM
            updm_v[pl.ds((bi * GB + r) * DIM, 16)] = gbuf_v[r, pl.ds(sub, 16)]
            return 0
        lax.fori_loop(0, GB, compact, 0)
        return 0
    lax.fori_loop(0, nb, gather_body, 0)

    magic = jnp.int32(0x5F3759DF)

    def clip_body(j, _):
        row = updm_v[pl.ds(j * DIM, 16)]
        ssq = jnp.sum(row * row)
        x = jnp.maximum(lax.broadcast_in_dim(ssq, (16,), ()),
                        jnp.float32(1e-24))
        y = plsc.bitcast(magic - (plsc.bitcast(x, jnp.int32) >> 1),
                         jnp.float32)
        y = y * (1.5 - 0.5 * x * y * y)
        y = y * (1.5 - 0.5 * x * y * y)
        y = y * (1.5 - 0.5 * x * y * y)
        scale = jnp.minimum(jnp.float32(1.0), y) * jnp.float32(-LR)
        updm_v[pl.ds(j * DIM, 16)] = row * scale
        return 0
    lax.fori_loop(0, cnt, clip_body, 0)

    nscan16 = (cnt + 15) // 16

    def build_sublist(lo, hi):
        def sub_body(t, k):
            lv = idxm_v[pl.ds(t * 16, 16)]
            jv = lanes + t * 16
            msk = (lv >= lo) & (lv < hi) & (jv < cnt)
            pf = plsc.cumsum(jnp.where(msk, jnp.int32(1), jnp.int32(0)))
            plsc.store_scatter(posc_v, [k + pf - 1], jv, mask=msk)
            return k + pf[15]
        return lax.fori_loop(0, nscan16, sub_body, jnp.int32(0))

    def apply_sublist(nmatch, lo, dst_ref):
        def app_body(j, _):
            p = posc_v[pl.ds(j, 16)][0]
            il = idxm_v[pl.ds(p, 16)][0] - lo
            ilv = lax.broadcast_in_dim(il, (16,), ())
            plsc.addupdate_scatter(dst_ref, [lanes, ilv],
                                   updm_v[pl.ds(p * DIM, 16)])
            return 0
        lax.fori_loop(0, nmatch, app_body, 0)

    def chunk_body(c, _):
        b = lax.rem(c, NSLOT)
        pltpu.make_async_copy(chunk_src(c), buf.at[b], sem_in.at[b]).wait()
        nmatch = build_sublist(c * ICH, (c + 1) * ICH)
        apply_sublist(nmatch, c * ICH, buf.at[b])
        pltpu.async_copy(buf.at[b], chunk_dst(c), sem_out.at[b])

        @pl.when(c >= 1)
        def _():
            bp = lax.rem(c - 1, NSLOT)
            pltpu.make_async_copy(buf.at[bp], chunk_dst(c - 1), sem_out.at[bp]).wait()

        @pl.when(c + 2 < NMAIN)
        def _():
            bn = lax.rem(c + 2, NSLOT)
            pltpu.async_copy(chunk_src(c + 2), buf.at[bn], sem_in.at[bn])
        return 0
    lax.fori_loop(0, NMAIN, chunk_body, 0)

    pltpu.make_async_copy(buf.at[(NMAIN - 1) % NSLOT], chunk_dst(NMAIN - 1),
                          sem_out.at[(NMAIN - 1) % NSLOT]).wait()

    @pl.when(wid < 4)
    def _():
        xs = pl.ds(ibase + I_MAIN, UT)
        pltpu.sync_copy(w_hbm.at[:, xs], xbuf)
        nmatch = build_sublist(I_MAIN, I_MAIN + UT)
        apply_sublist(nmatch, I_MAIN, xbuf)
        pltpu.sync_copy(xbuf, out_hbm.at[:, xs])

    @pl.when(wid == 4)
    def _():
        xs = pl.ds(TAIL_START, 64)
        pltpu.sync_copy(w_hbm.at[:, xs], xbuf2)
        nmatch = build_sublist(I_MAIN, I_MAIN + 64)
        apply_sublist(nmatch, I_MAIN, xbuf2)
        pltpu.sync_copy(xbuf2, out_hbm.at[:, xs])


def kernel(items_emb_weight, items_emb_grad, items):
    run = pl.kernel(
        _body,
        out_type=jax.ShapeDtypeStruct((DIM, M_ITEM), jnp.float32),
        mesh=plsc.VectorSubcoreMesh(core_axis_name="c", subcore_axis_name="s"),
        compiler_params=pltpu.CompilerParams(use_tc_tiling_on_sc=True,
                                             needs_layout_passes=False),
        scratch_types=[
            pltpu.VMEM((N_GRAD,), jnp.int32),
            pltpu.VMEM((CAP + 16,), jnp.int32),
            pltpu.VMEM((CAP + 16,), jnp.int32),
            pltpu.VMEM((CAP * DIM,), jnp.float32),
            pltpu.VMEM((CAP + 16,), jnp.int32),
            pltpu.VMEM((CAP + GB,), jnp.int32),
            pltpu.VMEM((GB, 128), jnp.float32),
            pltpu.VMEM((NSLOT, DIM, ICH), jnp.float32),
            pltpu.VMEM((DIM, UT), jnp.float32),
            pltpu.VMEM((DIM, 64), jnp.float32),
            pltpu.SemaphoreType.DMA((NSLOT,)),
            pltpu.SemaphoreType.DMA((NSLOT,)),
            pltpu.SemaphoreType.DMA,
        ],
    )
    out = run(items_emb_weight.T,
              items_emb_grad.reshape(N_GRAD // 8, 8 * DIM),
              items.astype(jnp.int32))
    return out.T
